# Initial kernel scaffold; baseline (speedup 1.0000x reference)
import jax
import jax.numpy as jnp
from jax.experimental import pallas as pl


def kernel(x, edge_index, W0, b0, g0, be0, Wl1, bl1, Wr1, Wl2, bl2, Wr2):
    raise NotImplementedError("write your pallas kernel here")


# speedup vs baseline: 3.9301x; 3.9301x over previous
import functools

import jax
import jax.numpy as jnp
from jax import lax
from jax.experimental import pallas as pl
from jax.experimental.pallas import tpu as pltpu
from jax.experimental.pallas import tpu_sc as plsc

_NC = 2
_NS = 16
_NW = _NC * _NS
_CH = 128
_LN_EPS = 1e-5


def _sc_agg(table, srcp, dstp, n_pad, with_cnt):
    e_pad = srcp.shape[0]
    epw = e_pad // _NW
    k_chunks = epw // _CH
    rows_per_tile = n_pad // _NS
    f = table.shape[1]

    mesh = plsc.VectorSubcoreMesh(
        core_axis_name="c", subcore_axis_name="s",
        num_cores=_NC, num_subcores=_NS)

    out_type = [jax.ShapeDtypeStruct((_NC, n_pad, f), jnp.float32)]
    scratch = [
        pltpu.VMEM((_CH,), jnp.int32),
        pltpu.VMEM((_CH,), jnp.int32),
        pltpu.VMEM((_CH, f), jnp.float32),
        pltpu.VMEM((16, f), jnp.float32),
        pltpu.VMEM_SHARED((n_pad, f), jnp.float32),
        pltpu.SemaphoreType.DMA,
    ]
    if with_cnt:
        out_type.append(jax.ShapeDtypeStruct((_NW * n_pad,), jnp.float32))
        scratch.append(pltpu.VMEM((n_pad,), jnp.float32))

    def body(table_hbm, src_hbm, dst_hbm, *rest):
        if with_cnt:
            (sum_out, cnt_out, src_v, dst_v, rows_v, zb_v, acc_sh, sem,
             lcnt_v) = rest
        else:
            sum_out, src_v, dst_v, rows_v, zb_v, acc_sh, sem = rest
        c = lax.axis_index("c")
        s = lax.axis_index("s")
        wid = s * _NC + c

        for r in range(16):
            for q in range(f // 16):
                zb_v[r, pl.ds(q * 16, 16)] = jnp.zeros((16,), jnp.float32)
        if with_cnt:
            def czero_body(i, _):
                lcnt_v[pl.ds(i * 16, 16)] = jnp.zeros((16,), jnp.float32)
                return 0
            lax.fori_loop(0, n_pad // 16, czero_body, 0)

        rbase = s * rows_per_tile
        nfull = rows_per_tile // 16
        rem = rows_per_tile - nfull * 16

        def zero_body(j, _):
            pltpu.sync_copy(zb_v, acc_sh.at[pl.ds(rbase + j * 16, 16)])
            return 0
        lax.fori_loop(0, nfull, zero_body, 0)
        if rem:
            pltpu.sync_copy(zb_v.at[pl.ds(0, rem)],
                            acc_sh.at[pl.ds(rbase + nfull * 16, rem)])
        plsc.subcore_barrier()

        ebase = wid * epw
        ones16 = jnp.ones((16,), jnp.float32)

        def edge_body(k, _):
            off = ebase + k * _CH
            pltpu.sync_copy(src_hbm.at[pl.ds(off, _CH)], src_v)
            pltpu.sync_copy(dst_hbm.at[pl.ds(off, _CH)], dst_v)
            pltpu.async_copy(table_hbm.at[src_v], rows_v, sem).wait()
            pltpu.sync_copy(rows_v, acc_sh.at[dst_v], add=True)
            if with_cnt:
                for j in range(_CH // 16):
                    dv = dst_v[pl.ds(j * 16, 16)]
                    plsc.addupdate_scatter(lcnt_v, [dv], ones16)
            return 0
        lax.fori_loop(0, k_chunks, edge_body, 0)
        plsc.subcore_barrier()

        pltpu.sync_copy(acc_sh.at[pl.ds(rbase, rows_per_tile)],
                        sum_out.at[c, pl.ds(rbase, rows_per_tile)])
        if with_cnt:
            pltpu.sync_copy(lcnt_v, cnt_out.at[pl.ds(wid * n_pad, n_pad)])

    kfn = pl.kernel(body, mesh=mesh, out_type=tuple(out_type),
                    scratch_types=scratch,
                    compiler_params=pltpu.CompilerParams(
                        needs_layout_passes=False))
    return kfn(table, srcp, dstp)


def _tc_proj_ln(x_p, w0t, b0, g0, be0):
    n_pad, f = x_p.shape
    h = w0t.shape[1]
    rb = n_pad // 4

    def body(x_ref, w_ref, b_ref, g_ref, be_ref, o_ref):
        z = jnp.dot(x_ref[...], w_ref[...],
                    preferred_element_type=jnp.float32) + b_ref[...]
        mu = jnp.mean(z, axis=-1, keepdims=True)
        var = jnp.mean((z - mu) ** 2, axis=-1, keepdims=True)
        y = (z - mu) / jnp.sqrt(var + _LN_EPS) * g_ref[...] + be_ref[...]
        o_ref[...] = jnp.maximum(y, 0.0)

    return pl.pallas_call(
        body,
        grid=(4,),
        in_specs=[
            pl.BlockSpec((rb, f), lambda i: (i, 0)),
            pl.BlockSpec((f, h), lambda i: (0, 0)),
            pl.BlockSpec((1, h), lambda i: (0, 0)),
            pl.BlockSpec((1, h), lambda i: (0, 0)),
            pl.BlockSpec((1, h), lambda i: (0, 0)),
        ],
        out_specs=pl.BlockSpec((rb, h), lambda i: (i, 0)),
        out_shape=jax.ShapeDtypeStruct((n_pad, h), jnp.float32),
    )(x_p, w0t, b0, g0, be0)


def _tc_sage(ssum, cnt, hin, wlt, bl, wrt, relu, resid):
    _, n_pad, f = ssum.shape
    h = wlt.shape[1]
    rb = n_pad // 4

    def body(s_ref, c_ref, h_ref, wl_ref, bl_ref, wr_ref, o_ref):
        stot = s_ref[0] + s_ref[1]
        ctot = jnp.sum(c_ref[...], axis=1, keepdims=True)
        mean = stot / jnp.maximum(ctot, 1.0)
        y = (jnp.dot(mean, wl_ref[...], preferred_element_type=jnp.float32)
             + bl_ref[...]
             + jnp.dot(h_ref[...], wr_ref[...],
                       preferred_element_type=jnp.float32))
        if resid:
            y = y + h_ref[...]
        if relu:
            y = jnp.maximum(y, 0.0)
        o_ref[...] = y

    return pl.pallas_call(
        body,
        grid=(4,),
        in_specs=[
            pl.BlockSpec((2, rb, f), lambda i: (0, i, 0)),
            pl.BlockSpec((rb, _NW), lambda i: (i, 0)),
            pl.BlockSpec((rb, f), lambda i: (i, 0)),
            pl.BlockSpec((f, h), lambda i: (0, 0)),
            pl.BlockSpec((1, h), lambda i: (0, 0)),
            pl.BlockSpec((f, h), lambda i: (0, 0)),
        ],
        out_specs=pl.BlockSpec((rb, h), lambda i: (i, 0)),
        out_shape=jax.ShapeDtypeStruct((n_pad, h), jnp.float32),
    )(ssum, cnt, hin, wlt, bl, wrt)


def kernel(x, edge_index, W0, b0, g0, be0, Wl1, bl1, Wr1, Wl2, bl2, Wr2):
    n, f = x.shape
    e = edge_index.shape[1]

    n_pad = ((n + 1 + 127) // 128) * 128
    chunk = _CH * _NW
    e_pad = ((e + chunk - 1) // chunk) * chunk

    src = edge_index[0]
    dst = edge_index[1]
    srcp = jnp.concatenate([src, jnp.zeros((e_pad - e,), jnp.int32)])
    dstp = jnp.concatenate([dst, jnp.full((e_pad - e,), n, jnp.int32)])
    x_p = jnp.concatenate([x, jnp.zeros((n_pad - n, f), x.dtype)])

    h1 = _tc_proj_ln(x_p, W0.T, b0.reshape(1, -1), g0.reshape(1, -1),
                     be0.reshape(1, -1))

    s1, cnt_flat = _sc_agg(h1, srcp, dstp, n_pad, with_cnt=True)
    cnt = cnt_flat.reshape(_NW, n_pad).T
    x1 = _tc_sage(s1, cnt, h1, Wl1.T, bl1.reshape(1, -1), Wr1.T,
                  relu=True, resid=False)

    (s2,) = _sc_agg(x1, srcp, dstp, n_pad, with_cnt=False)
    out_p = _tc_sage(s2, cnt, x1, Wl2.T, bl2.reshape(1, -1), Wr2.T,
                     relu=False, resid=True)

    return out_p[:n]
